# baseline (speedup 1.0000x reference)
import functools

import jax
import jax.numpy as jnp
from jax import lax
from jax.experimental import pallas as pl
from jax.experimental.pallas import tpu as pltpu
from jax.experimental.pallas import tpu_sc as plsc

_B = 4096
_CTX = 20
_D = 64
_NS = 5
_NW = 32
_BPW = _B // _NW
_NPW = _BPW * _NS
_ECH = 32
_NCH = _BPW // _ECH
_CROWS = _ECH * _CTX
_IDXCAP = 128


def _pool_body(inputs_hbm, target_hbm, nwords_hbm, lecture_hbm, word_hbm,
               d_out, t_out, n_out,
               ids_v, tid_v, nid_v, did_v, cid_v,
               crow_v, acc_v, drow_v, trow_v, nrow_v,
               sem_in, sem_doc, sem_tgt, sem_neg, sem_ctx):
    wid = lax.axis_index("s") * 2 + lax.axis_index("c")
    base = wid * _BPW

    pltpu.sync_copy(inputs_hbm.at[pl.ds(base * (_CTX + 1),
                                        _BPW * (_CTX + 1))], ids_v)
    cp_t = pltpu.async_copy(target_hbm.at[pl.ds(base, _BPW)], tid_v, sem_in)
    cp_n = pltpu.async_copy(nwords_hbm.at[pl.ds(base * _NS, _NPW)], nid_v,
                            sem_in)

    for v in range(_BPW // 16):
        rows = (lax.iota(jnp.int32, 16) + v * 16) * (_CTX + 1)
        did_v[pl.ds(v * 16, 16)] = plsc.load_gather(ids_v, [rows])

    doc_cp = pltpu.async_copy(lecture_hbm.at[did_v], drow_v, sem_doc)
    cp_t.wait()
    tgt_cp = pltpu.async_copy(word_hbm.at[tid_v], trow_v, sem_tgt)
    cp_n.wait()
    neg_cps = [
        pltpu.async_copy(word_hbm.at[nid_v.at[pl.ds(k * _IDXCAP, _IDXCAP)]],
                         nrow_v.at[pl.ds(k * _IDXCAP, _IDXCAP)], sem_neg)
        for k in range(_NPW // _IDXCAP)
    ]
    doc_cp.wait()

    for ch in range(_NCH):
        for v in range(_CROWS // 16):
            p = lax.iota(jnp.int32, 16) + v * 16
            ex = p // _CTX
            pos = (ex + ch * _ECH) * (_CTX + 1) + (p - ex * _CTX) + 1
            cid_v[pl.ds(v * 16, 16)] = plsc.load_gather(ids_v, [pos])
        ctx_cps = [
            pltpu.async_copy(
                word_hbm.at[cid_v.at[pl.ds(k * _IDXCAP, _IDXCAP)]],
                crow_v.at[pl.ds(k * _IDXCAP, _IDXCAP)], sem_ctx)
            for k in range(_CROWS // _IDXCAP)
        ]
        for cp in ctx_cps:
            cp.wait()

        def ex_body(el, carry, ch=ch):
            e = ch * _ECH + el
            for dv in range(_D // 16):
                sl = pl.ds(dv * 16, 16)
                a = drow_v[e, sl]
                for j in range(_CTX):
                    a = a + crow_v[el * _CTX + j, sl]
                acc_v[e, sl] = a
            return carry

        lax.fori_loop(0, _ECH, ex_body, 0)

    def scale_body(e, carry):
        for dv in range(_D // 16):
            sl = pl.ds(dv * 16, 16)
            acc_v[e, sl] = acc_v[e, sl] * jnp.float32(1.0 / 21.0)
        return carry

    lax.fori_loop(0, _BPW, scale_body, 0)
    pltpu.sync_copy(acc_v, d_out.at[pl.ds(base, _BPW)])

    tgt_cp.wait()
    pltpu.sync_copy(trow_v, t_out.at[pl.ds(base, _BPW)])

    for cp in neg_cps:
        cp.wait()

    def neg_body(r, carry):
        for dv in range(_D // 16):
            sl = pl.ds(dv * 16, 16)
            nrow_v[r, sl] = -nrow_v[r, sl]
        return carry

    lax.fori_loop(0, _NPW, neg_body, 0)
    pltpu.sync_copy(nrow_v, n_out.at[pl.ds(base * _NS, _NPW)])


@jax.jit
def _sc_gather_pool(inputs, target, nwords, lecture, word_emb):
    mesh = plsc.VectorSubcoreMesh(core_axis_name="c", subcore_axis_name="s")
    kfn = pl.kernel(
        _pool_body,
        out_type=(
            jax.ShapeDtypeStruct((_B, _D), jnp.float32),
            jax.ShapeDtypeStruct((_B, _D), jnp.float32),
            jax.ShapeDtypeStruct((_B * _NS, _D), jnp.float32),
        ),
        mesh=mesh,
        compiler_params=pltpu.CompilerParams(needs_layout_passes=False,
                                             use_tc_tiling_on_sc=False),
        scratch_types=[
            pltpu.VMEM((_BPW * (_CTX + 1),), jnp.int32),
            pltpu.VMEM((_BPW,), jnp.int32),
            pltpu.VMEM((_NPW,), jnp.int32),
            pltpu.VMEM((_BPW,), jnp.int32),
            pltpu.VMEM((_CROWS,), jnp.int32),
            pltpu.VMEM((_CROWS, _D), jnp.float32),
            pltpu.VMEM((_BPW, _D), jnp.float32),
            pltpu.VMEM((_BPW, _D), jnp.float32),
            pltpu.VMEM((_BPW, _D), jnp.float32),
            pltpu.VMEM((_NPW, _D), jnp.float32),
            pltpu.SemaphoreType.DMA,
            pltpu.SemaphoreType.DMA,
            pltpu.SemaphoreType.DMA,
            pltpu.SemaphoreType.DMA,
            pltpu.SemaphoreType.DMA,
        ],
    )
    return kfn(inputs, target, nwords, lecture, word_emb)


def kernel(inputs, target, lecture, word_emb, freq_dic):
    bsz = target.shape[0]
    cdf = jnp.cumsum(freq_dic)
    u = jax.random.uniform(jax.random.key(42), (bsz * _NS,),
                           dtype=jnp.float32) * cdf[-1]
    nwords = jnp.clip(jnp.searchsorted(cdf, u), 0,
                      freq_dic.shape[0] - 1).astype(jnp.int32)

    d, t, n = _sc_gather_pool(inputs.astype(jnp.int32).reshape(-1),
                              target.astype(jnp.int32),
                              nwords, lecture, word_emb)
    return (d[:, None, :], t[:, None, :], n.reshape(bsz, _D, _NS))


# speedup vs baseline: 1.2045x; 1.2045x over previous
import functools

import jax
import jax.numpy as jnp
from jax import lax
from jax.experimental import pallas as pl
from jax.experimental.pallas import tpu as pltpu
from jax.experimental.pallas import tpu_sc as plsc

_B = 4096
_CTX = 20
_D = 64
_NS = 5
_V = 1000000
_NW = 32
_BPW = _B // _NW
_NPW = _BPW * _NS
_QV = _NPW // 16
_ECH = 16
_NCH = _BPW // _ECH
_CROWS = _ECH * _CTX
_IDXCAP = 128
_LEVELS = 20


def _body(inputs_hbm, target_hbm, u_hbm, cdf_hbm, lecture_hbm, word_hbm,
          d_out, t_out, n_out,
          ids_v, tid_v, did_v, cid_v,
          u_v, lo_v, hi_v, mid_v, val_v, nidx_v,
          crow_v, acc_v, drow_v, trow_v, nrow_v,
          sem_in, sem_doc, sem_tgt, sem_neg, sem_ctx0, sem_ctx1, sem_s):
    wid = lax.axis_index("s") * 2 + lax.axis_index("c")
    base = wid * _BPW
    sem_ctx = (sem_ctx0, sem_ctx1)

    def build_cid(ch, buf):
        for v in range(_CROWS // 16):
            p = lax.iota(jnp.int32, 16) + v * 16
            ex = p // _CTX
            pos = (ex + ch * _ECH) * (_CTX + 1) + (p - ex * _CTX) + 1
            cid_v[buf, v // 8, pl.ds((v % 8) * 16, 16)] = plsc.load_gather(
                ids_v, [pos])

    def fire_ctx(buf):
        return [
            pltpu.async_copy(word_hbm.at[cid_v.at[buf, 0]],
                             crow_v.at[buf, pl.ds(0, 128)], sem_ctx[buf]),
            pltpu.async_copy(word_hbm.at[cid_v.at[buf, 1]],
                             crow_v.at[buf, pl.ds(128, 128)], sem_ctx[buf]),
            pltpu.async_copy(word_hbm.at[cid_v.at[buf, 2, pl.ds(0, 64)]],
                             crow_v.at[buf, pl.ds(256, 64)], sem_ctx[buf]),
        ]

    pltpu.sync_copy(inputs_hbm.at[pl.ds(base * (_CTX + 1),
                                        _BPW * (_CTX + 1))], ids_v)
    cp_t = pltpu.async_copy(target_hbm.at[pl.ds(base, _BPW)], tid_v, sem_in)
    cp_u = pltpu.async_copy(u_hbm.at[wid], u_v, sem_in)

    for v in range(_BPW // 16):
        rows = (lax.iota(jnp.int32, 16) + v * 16) * (_CTX + 1)
        did_v[pl.ds(v * 16, 16)] = plsc.load_gather(ids_v, [rows])
    doc_cp = pltpu.async_copy(lecture_hbm.at[did_v], drow_v, sem_doc)
    cp_t.wait()
    tgt_cp = pltpu.async_copy(word_hbm.at[tid_v], trow_v, sem_tgt)

    build_cid(0, 0)
    cps0 = fire_ctx(0)
    build_cid(1, 1)
    cps1 = fire_ctx(1)
    ctx_cps = [cps0, cps1]
    cp_u.wait()

    def init_body(v, carry):
        lo_v[v] = jnp.zeros((16,), jnp.int32)
        hi_v[v] = jnp.full((16,), _V, jnp.int32)
        return carry

    lax.fori_loop(0, _QV, init_body, 0)

    for _ in range(_LEVELS):
        def mid_body(k, carry):
            for j in range(8):
                v = k * 8 + j
                lo = lo_v[v]
                mid_v[k, pl.ds(j * 16, 16)] = lo + lax.shift_right_logical(
                    hi_v[v] - lo, 1)
            return carry

        lax.fori_loop(0, _QV // 8, mid_body, 0)
        s_cps = [
            pltpu.async_copy(cdf_hbm.at[mid_v.at[k]], val_v.at[k], sem_s)
            for k in range(_QV // 8)
        ]
        for cp in s_cps:
            cp.wait()

        def upd_body(k, carry):
            for j in range(8):
                v = k * 8 + j
                sl = pl.ds(j * 16, 16)
                pred = u_v[v] <= val_v[k, sl]
                mid = mid_v[k, sl]
                hi_v[v] = jnp.where(pred, mid, hi_v[v])
                lo_v[v] = jnp.where(pred, lo_v[v], mid)
            return carry

        lax.fori_loop(0, _QV // 8, upd_body, 0)

    for k in range(_NPW // _IDXCAP):
        for j in range(8):
            nidx_v[k, pl.ds(j * 16, 16)] = jnp.minimum(
                hi_v[k * 8 + j], jnp.int32(_V - 1))
    neg_cps = [
        pltpu.async_copy(word_hbm.at[nidx_v.at[k]],
                         nrow_v.at[pl.ds(k * _IDXCAP, _IDXCAP)], sem_neg)
        for k in range(_NPW // _IDXCAP)
    ]
    doc_cp.wait()

    for ch in range(_NCH):
        buf = ch % 2
        for cp in ctx_cps[buf]:
            cp.wait()

        def ex_body(el, carry, ch=ch, buf=buf):
            e = ch * _ECH + el
            for dv in range(_D // 16):
                sl = pl.ds(dv * 16, 16)
                a = drow_v[e, sl]
                for j in range(_CTX):
                    a = a + crow_v[buf, el * _CTX + j, sl]
                acc_v[e, sl] = a
            return carry

        lax.fori_loop(0, _ECH, ex_body, 0)
        if ch + 2 < _NCH:
            build_cid(ch + 2, buf)
            ctx_cps[buf] = fire_ctx(buf)

    def scale_body(e, carry):
        for dv in range(_D // 16):
            sl = pl.ds(dv * 16, 16)
            acc_v[e, sl] = acc_v[e, sl] * jnp.float32(1.0 / 21.0)
        return carry

    lax.fori_loop(0, _BPW, scale_body, 0)
    pltpu.sync_copy(acc_v, d_out.at[pl.ds(base, _BPW)])

    tgt_cp.wait()
    pltpu.sync_copy(trow_v, t_out.at[pl.ds(base, _BPW)])

    for cp in neg_cps:
        cp.wait()

    def neg_body(r, carry):
        for dv in range(_D // 16):
            sl = pl.ds(dv * 16, 16)
            nrow_v[r, sl] = -nrow_v[r, sl]
        return carry

    lax.fori_loop(0, _NPW, neg_body, 0)
    pltpu.sync_copy(nrow_v, n_out.at[pl.ds(base * _NS, _NPW)])


@jax.jit
def _sc_doc2vec(inputs, target, u, cdf, lecture, word_emb):
    mesh = plsc.VectorSubcoreMesh(core_axis_name="c", subcore_axis_name="s")
    kfn = pl.kernel(
        _body,
        out_type=(
            jax.ShapeDtypeStruct((_B, _D), jnp.float32),
            jax.ShapeDtypeStruct((_B, _D), jnp.float32),
            jax.ShapeDtypeStruct((_B * _NS, _D), jnp.float32),
        ),
        mesh=mesh,
        compiler_params=pltpu.CompilerParams(needs_layout_passes=False,
                                             use_tc_tiling_on_sc=False),
        scratch_types=[
            pltpu.VMEM((_BPW * (_CTX + 1),), jnp.int32),
            pltpu.VMEM((_BPW,), jnp.int32),
            pltpu.VMEM((_BPW,), jnp.int32),
            pltpu.VMEM((2, 3, _IDXCAP), jnp.int32),
            pltpu.VMEM((_QV, 16), jnp.float32),
            pltpu.VMEM((_QV, 16), jnp.int32),
            pltpu.VMEM((_QV, 16), jnp.int32),
            pltpu.VMEM((_QV // 8, 128), jnp.int32),
            pltpu.VMEM((_QV // 8, 128), jnp.float32),
            pltpu.VMEM((_NPW // _IDXCAP, _IDXCAP), jnp.int32),
            pltpu.VMEM((2, _CROWS, _D), jnp.float32),
            pltpu.VMEM((_BPW, _D), jnp.float32),
            pltpu.VMEM((_BPW, _D), jnp.float32),
            pltpu.VMEM((_BPW, _D), jnp.float32),
            pltpu.VMEM((_NPW, _D), jnp.float32),
            pltpu.SemaphoreType.DMA,
            pltpu.SemaphoreType.DMA,
            pltpu.SemaphoreType.DMA,
            pltpu.SemaphoreType.DMA,
            pltpu.SemaphoreType.DMA,
            pltpu.SemaphoreType.DMA,
            pltpu.SemaphoreType.DMA,
        ],
    )
    return kfn(inputs, target, u, cdf, lecture, word_emb)


def kernel(inputs, target, lecture, word_emb, freq_dic):
    bsz = target.shape[0]
    cdf = jnp.cumsum(freq_dic)
    u = jax.random.uniform(jax.random.key(42), (bsz * _NS,),
                           dtype=jnp.float32) * cdf[-1]

    d, t, n = _sc_doc2vec(inputs.astype(jnp.int32).reshape(-1),
                          target.astype(jnp.int32),
                          u.reshape(_NW, _QV, 16), cdf, lecture, word_emb)
    return (d[:, None, :], t[:, None, :], n.reshape(bsz, _D, _NS))


# speedup vs baseline: 1.4166x; 1.1761x over previous
import functools

import jax
import jax.numpy as jnp
from jax import lax
from jax.experimental import pallas as pl
from jax.experimental.pallas import tpu as pltpu
from jax.experimental.pallas import tpu_sc as plsc

_B = 4096
_CTX = 20
_D = 64
_NS = 5
_V = 1000000
_NW = 32
_BPW = _B // _NW
_NPW = _BPW * _NS
_QV = _NPW // 16
_ECH = 16
_NCH = _BPW // _ECH
_CROWS = _ECH * _CTX
_IDXCAP = 128
_S = 128
_NG = 8192
_GL = 14
_HL = 7


def _body(inputs_hbm, target_hbm, u_hbm, cdf_hbm, grid_hbm, lecture_hbm,
          word_hbm,
          d_out, t_out, n_out,
          ids_v, tid_v, did_v, cid_v,
          u_v, lo_v, hi_v, mid_v, val_v, nidx_v, grid_v,
          crow_v, acc_v, drow_v, trow_v, nrow_v,
          sem_in, sem_doc, sem_tgt, sem_neg, sem_ctx0, sem_ctx1, sem_s):
    wid = lax.axis_index("s") * 2 + lax.axis_index("c")
    base = wid * _BPW
    sem_ctx = (sem_ctx0, sem_ctx1)

    def build_cid(ch, buf):
        for v in range(_CROWS // 16):
            p = lax.iota(jnp.int32, 16) + v * 16
            ex = p // _CTX
            pos = (ex + ch * _ECH) * (_CTX + 1) + (p - ex * _CTX) + 1
            cid_v[buf, v // 8, pl.ds((v % 8) * 16, 16)] = plsc.load_gather(
                ids_v, [pos])

    def fire_ctx(buf):
        return [
            pltpu.async_copy(word_hbm.at[cid_v.at[buf, 0]],
                             crow_v.at[buf, pl.ds(0, 128)], sem_ctx[buf]),
            pltpu.async_copy(word_hbm.at[cid_v.at[buf, 1]],
                             crow_v.at[buf, pl.ds(128, 128)], sem_ctx[buf]),
            pltpu.async_copy(word_hbm.at[cid_v.at[buf, 2, pl.ds(0, 64)]],
                             crow_v.at[buf, pl.ds(256, 64)], sem_ctx[buf]),
        ]

    pltpu.sync_copy(inputs_hbm.at[pl.ds(base * (_CTX + 1),
                                        _BPW * (_CTX + 1))], ids_v)
    cp_t = pltpu.async_copy(target_hbm.at[pl.ds(base, _BPW)], tid_v, sem_in)
    cp_u = pltpu.async_copy(u_hbm.at[wid], u_v, sem_in)
    cp_g = pltpu.async_copy(grid_hbm, grid_v, sem_in)

    for v in range(_BPW // 16):
        rows = (lax.iota(jnp.int32, 16) + v * 16) * (_CTX + 1)
        did_v[pl.ds(v * 16, 16)] = plsc.load_gather(ids_v, [rows])
    doc_cp = pltpu.async_copy(lecture_hbm.at[did_v], drow_v, sem_doc)
    cp_t.wait()
    tgt_cp = pltpu.async_copy(word_hbm.at[tid_v], trow_v, sem_tgt)

    build_cid(0, 0)
    cps0 = fire_ctx(0)
    build_cid(1, 1)
    cps1 = fire_ctx(1)
    ctx_cps = [cps0, cps1]
    cp_u.wait()

    cp_g.wait()

    def init_body(v, carry):
        lo_v[v] = jnp.zeros((16,), jnp.int32)
        hi_v[v] = jnp.full((16,), _NG, jnp.int32)
        return carry

    lax.fori_loop(0, _QV, init_body, 0)

    for _ in range(_GL):
        def grid_body(v, carry):
            lo = lo_v[v]
            mid = lo + lax.shift_right_logical(hi_v[v] - lo, 1)
            pred = u_v[v] <= plsc.load_gather(grid_v, [mid])
            hi_v[v] = jnp.where(pred, mid, hi_v[v])
            lo_v[v] = jnp.where(pred, lo, mid)
            return carry

        lax.fori_loop(0, _QV, grid_body, 0)

    def bucket_body(v, carry):
        b = hi_v[v] * _S
        lo_v[v] = jnp.minimum(jnp.maximum(b - 1, 0), jnp.int32(_V - 1))
        hi_v[v] = jnp.minimum(b + (_S - 1), jnp.int32(_V))
        return carry

    lax.fori_loop(0, _QV, bucket_body, 0)

    for _ in range(_HL):
        def mid_body(k, carry):
            for j in range(8):
                v = k * 8 + j
                lo = lo_v[v]
                mid_v[k, pl.ds(j * 16, 16)] = lo + lax.shift_right_logical(
                    hi_v[v] - lo, 1)
            return carry

        lax.fori_loop(0, _QV // 8, mid_body, 0)
        s_cps = [
            pltpu.async_copy(cdf_hbm.at[mid_v.at[k]], val_v.at[k], sem_s)
            for k in range(_QV // 8)
        ]
        for cp in s_cps:
            cp.wait()

        def upd_body(k, carry):
            for j in range(8):
                v = k * 8 + j
                sl = pl.ds(j * 16, 16)
                pred = u_v[v] <= val_v[k, sl]
                mid = mid_v[k, sl]
                hi_v[v] = jnp.where(pred, mid, hi_v[v])
                lo_v[v] = jnp.where(pred, lo_v[v], mid)
            return carry

        lax.fori_loop(0, _QV // 8, upd_body, 0)

    for k in range(_NPW // _IDXCAP):
        for j in range(8):
            nidx_v[k, pl.ds(j * 16, 16)] = jnp.minimum(
                hi_v[k * 8 + j], jnp.int32(_V - 1))
    neg_cps = [
        pltpu.async_copy(word_hbm.at[nidx_v.at[k]],
                         nrow_v.at[pl.ds(k * _IDXCAP, _IDXCAP)], sem_neg)
        for k in range(_NPW // _IDXCAP)
    ]
    doc_cp.wait()

    for ch in range(_NCH):
        buf = ch % 2
        for cp in ctx_cps[buf]:
            cp.wait()

        def ex_body(el, carry, ch=ch, buf=buf):
            e = ch * _ECH + el
            for dv in range(_D // 16):
                sl = pl.ds(dv * 16, 16)
                a = drow_v[e, sl]
                for j in range(_CTX):
                    a = a + crow_v[buf, el * _CTX + j, sl]
                acc_v[e, sl] = a
            return carry

        lax.fori_loop(0, _ECH, ex_body, 0)
        if ch + 2 < _NCH:
            build_cid(ch + 2, buf)
            ctx_cps[buf] = fire_ctx(buf)

    def scale_body(e, carry):
        for dv in range(_D // 16):
            sl = pl.ds(dv * 16, 16)
            acc_v[e, sl] = acc_v[e, sl] * jnp.float32(1.0 / 21.0)
        return carry

    lax.fori_loop(0, _BPW, scale_body, 0)
    pltpu.sync_copy(acc_v, d_out.at[pl.ds(base, _BPW)])

    tgt_cp.wait()
    pltpu.sync_copy(trow_v, t_out.at[pl.ds(base, _BPW)])

    for cp in neg_cps:
        cp.wait()

    def neg_body(r, carry):
        for dv in range(_D // 16):
            sl = pl.ds(dv * 16, 16)
            nrow_v[r, sl] = -nrow_v[r, sl]
        return carry

    lax.fori_loop(0, _NPW, neg_body, 0)
    pltpu.sync_copy(nrow_v, n_out.at[pl.ds(base * _NS, _NPW)])


@jax.jit
def _sc_doc2vec(inputs, target, u, cdf, grid, lecture, word_emb):
    mesh = plsc.VectorSubcoreMesh(core_axis_name="c", subcore_axis_name="s")
    kfn = pl.kernel(
        _body,
        out_type=(
            jax.ShapeDtypeStruct((_B, _D), jnp.float32),
            jax.ShapeDtypeStruct((_B, _D), jnp.float32),
            jax.ShapeDtypeStruct((_B * _NS, _D), jnp.float32),
        ),
        mesh=mesh,
        compiler_params=pltpu.CompilerParams(needs_layout_passes=False,
                                             use_tc_tiling_on_sc=False),
        scratch_types=[
            pltpu.VMEM((_BPW * (_CTX + 1),), jnp.int32),
            pltpu.VMEM((_BPW,), jnp.int32),
            pltpu.VMEM((_BPW,), jnp.int32),
            pltpu.VMEM((2, 3, _IDXCAP), jnp.int32),
            pltpu.VMEM((_QV, 16), jnp.float32),
            pltpu.VMEM((_QV, 16), jnp.int32),
            pltpu.VMEM((_QV, 16), jnp.int32),
            pltpu.VMEM((_QV // 8, 128), jnp.int32),
            pltpu.VMEM((_QV // 8, 128), jnp.float32),
            pltpu.VMEM((_NPW // _IDXCAP, _IDXCAP), jnp.int32),
            pltpu.VMEM((_NG,), jnp.float32),
            pltpu.VMEM((2, _CROWS, _D), jnp.float32),
            pltpu.VMEM((_BPW, _D), jnp.float32),
            pltpu.VMEM((_BPW, _D), jnp.float32),
            pltpu.VMEM((_BPW, _D), jnp.float32),
            pltpu.VMEM((_NPW, _D), jnp.float32),
            pltpu.SemaphoreType.DMA,
            pltpu.SemaphoreType.DMA,
            pltpu.SemaphoreType.DMA,
            pltpu.SemaphoreType.DMA,
            pltpu.SemaphoreType.DMA,
            pltpu.SemaphoreType.DMA,
            pltpu.SemaphoreType.DMA,
        ],
    )
    return kfn(inputs, target, u, cdf, grid, lecture, word_emb)


def kernel(inputs, target, lecture, word_emb, freq_dic):
    bsz = target.shape[0]
    cdf = jnp.cumsum(freq_dic)
    u = jax.random.uniform(jax.random.key(42), (bsz * _NS,),
                           dtype=jnp.float32) * cdf[-1]
    grid = cdf[jnp.minimum(
        jnp.arange(_NG, dtype=jnp.int32) * _S + (_S - 1), _V - 1)]

    d, t, n = _sc_doc2vec(inputs.astype(jnp.int32).reshape(-1),
                          target.astype(jnp.int32),
                          u.reshape(_NW, _QV, 16), cdf, grid,
                          lecture, word_emb)
    return (d[:, None, :], t[:, None, :], n.reshape(bsz, _D, _NS))
